# Initial kernel scaffold; baseline (speedup 1.0000x reference)
import functools

import jax
import jax.numpy as jnp
from jax import lax
from jax.experimental import pallas as pl
from jax.experimental.pallas import tpu as pltpu
from jax.experimental.pallas import tpu_sc as plsc

N = 10000
D = 128
E = 320000
NC = 2
NS = 16
NW = NC * NS
EPT = E // NW
CHUNK = 80
NCHUNK = EPT // CHUNK
GROUPS = CHUNK // 16
RPS = N // NS
ZROWS = 125
ROWB = 1000

_f32 = jnp.float32
_i32 = jnp.int32


def _vmesh():
    return plsc.VectorSubcoreMesh(core_axis_name="c", subcore_axis_name="s")


def _deg_partials(w3, dst3):
    @functools.partial(
        pl.kernel,
        out_type=jax.ShapeDtypeStruct((NC, N, 16), _f32),
        mesh=_vmesh(),
        scratch_types=[
            pltpu.VMEM((EPT,), _f32),
            pltpu.VMEM((NCHUNK, CHUNK), _i32),
            pltpu.VMEM((CHUNK, 16), _f32),
            pltpu.VMEM((ZROWS, 16), _f32),
            pltpu.VMEM_SHARED((N, 16), _f32),
        ],
    )
    def k(w_hbm, dst_hbm, out_hbm, w_v, dst_v, rows_v, zb_v, acc):
        cid = lax.axis_index("c")
        sid = lax.axis_index("s")
        wid = sid * NC + cid
        zero16 = jnp.zeros((16,), _f32)
        for r in range(ZROWS):
            zb_v[r, :] = zero16
        for t in range(RPS // ZROWS):
            pltpu.sync_copy(zb_v, acc.at[pl.ds(sid * RPS + t * ZROWS, ZROWS)])
        pltpu.sync_copy(w_hbm.at[wid], w_v)
        pltpu.sync_copy(dst_hbm.at[wid], dst_v)
        plsc.subcore_barrier()

        @pl.loop(0, NCHUNK)
        def _(c):
            for g in range(GROUPS):
                wv = w_v[pl.ds(c * CHUNK + g * 16, 16)]
                for j in range(16):
                    rows_v[g * 16 + j, :] = jnp.take(
                        wv, jnp.full((16,), j, _i32), mode="promise_in_bounds")
            pltpu.sync_copy(rows_v, acc.at[dst_v.at[c]], add=True)

        plsc.subcore_barrier()
        for t in range(RPS // ZROWS):
            base = sid * RPS + t * ZROWS
            pltpu.sync_copy(acc.at[pl.ds(base, ZROWS)],
                            out_hbm.at[cid, pl.ds(base, ZROWS)])

    return k(w3, dst3)


def _agg_partials(g, src3, dst3, w3):
    @functools.partial(
        pl.kernel,
        out_type=jax.ShapeDtypeStruct((NC, N, D), _f32),
        mesh=_vmesh(),
        scratch_types=[
            pltpu.VMEM((EPT,), _i32),
            pltpu.VMEM((NCHUNK, CHUNK), _i32),
            pltpu.VMEM((EPT,), _f32),
            pltpu.VMEM((CHUNK, D), _f32),
            pltpu.VMEM((ZROWS, D), _f32),
            pltpu.VMEM_SHARED((N, D), _f32),
            pltpu.SemaphoreType.DMA,
        ],
    )
    def k(g_hbm, src_hbm, dst_hbm, w_hbm, out_hbm,
          src_v, dst_v, w_v, rows_v, zb_v, acc, sem):
        cid = lax.axis_index("c")
        sid = lax.axis_index("s")
        wid = sid * NC + cid
        zero16 = jnp.zeros((16,), _f32)
        for r in range(ZROWS):
            for cc in range(D // 16):
                zb_v[r, pl.ds(cc * 16, 16)] = zero16
        for t in range(RPS // ZROWS):
            pltpu.sync_copy(zb_v, acc.at[pl.ds(sid * RPS + t * ZROWS, ZROWS)])
        pltpu.sync_copy(src_hbm.at[wid], src_v)
        pltpu.sync_copy(dst_hbm.at[wid], dst_v)
        pltpu.sync_copy(w_hbm.at[wid], w_v)
        plsc.subcore_barrier()

        @pl.loop(0, NCHUNK)
        def _(c):
            pltpu.async_copy(
                g_hbm.at[src_v.at[pl.ds(c * CHUNK, CHUNK)]], rows_v, sem).wait()
            for g_ in range(GROUPS):
                wv = w_v[pl.ds(c * CHUNK + g_ * 16, 16)]
                for j in range(16):
                    sv = jnp.take(wv, jnp.full((16,), j, _i32),
                                  mode="promise_in_bounds")
                    row = g_ * 16 + j
                    for cc in range(D // 16):
                        sl = pl.ds(cc * 16, 16)
                        rows_v[row, sl] = rows_v[row, sl] * sv
            pltpu.sync_copy(rows_v, acc.at[dst_v.at[c]], add=True)

        plsc.subcore_barrier()
        for t in range(RPS // ZROWS):
            base = sid * RPS + t * ZROWS
            pltpu.sync_copy(acc.at[pl.ds(base, ZROWS)],
                            out_hbm.at[cid, pl.ds(base, ZROWS)])

    return k(g, src3, dst3, w3)


def _row_block(i):
    return (i, 0)


def _mm_kernel(x_ref, w_ref, o_ref):
    o_ref[...] = jnp.dot(x_ref[...], w_ref[...], preferred_element_type=_f32)


def _mm(x, w):
    return pl.pallas_call(
        _mm_kernel,
        grid=(N // ROWB,),
        in_specs=[pl.BlockSpec((ROWB, D), _row_block),
                  pl.BlockSpec((D, D), lambda i: (0, 0))],
        out_specs=pl.BlockSpec((ROWB, D), _row_block),
        out_shape=jax.ShapeDtypeStruct((N, D), _f32),
    )(x, w)


def _dinv_of(degp_ref):
    deg = degp_ref[0, :, 0:1] + degp_ref[1, :, 0:1] + 1.0
    return lax.rsqrt(deg)


def _scale_kernel(degp_ref, h_ref, o_ref):
    o_ref[...] = _dinv_of(degp_ref) * h_ref[...]


def _scale(degp, h):
    return pl.pallas_call(
        _scale_kernel,
        grid=(N // ROWB,),
        in_specs=[pl.BlockSpec((NC, ROWB, 16), lambda i: (0, i, 0)),
                  pl.BlockSpec((ROWB, D), _row_block)],
        out_specs=pl.BlockSpec((ROWB, D), _row_block),
        out_shape=jax.ShapeDtypeStruct((N, D), _f32),
    )(degp, h)


def _mid_kernel(degp_ref, p_ref, g_ref, b_ref, w_ref, o_ref):
    dinv = _dinv_of(degp_ref)
    z = dinv * (p_ref[0] + p_ref[1] + g_ref[...]) + b_ref[...]
    z = jnp.maximum(z, 0.0)
    o_ref[...] = dinv * jnp.dot(z, w_ref[...], preferred_element_type=_f32)


def _mid_layer(degp, p, g, b, w):
    return pl.pallas_call(
        _mid_kernel,
        grid=(N // ROWB,),
        in_specs=[pl.BlockSpec((NC, ROWB, 16), lambda i: (0, i, 0)),
                  pl.BlockSpec((NC, ROWB, D), lambda i: (0, i, 0)),
                  pl.BlockSpec((ROWB, D), _row_block),
                  pl.BlockSpec((1, D), lambda i: (0, 0)),
                  pl.BlockSpec((D, D), lambda i: (0, 0))],
        out_specs=pl.BlockSpec((ROWB, D), _row_block),
        out_shape=jax.ShapeDtypeStruct((N, D), _f32),
    )(degp, p, g, b.reshape(1, D), w)


def _final_kernel(degp_ref, p_ref, g_ref, b_ref, w_ref, b3_ref, o_ref):
    dinv = _dinv_of(degp_ref)
    z = dinv * (p_ref[0] + p_ref[1] + g_ref[...]) + b_ref[...]
    z = jnp.maximum(z, 0.0)
    o_ref[...] = (jnp.dot(z, w_ref[...], preferred_element_type=_f32)
                  + b3_ref[...])


def _final_layer(degp, p, g, b, w, b3):
    return pl.pallas_call(
        _final_kernel,
        grid=(N // ROWB,),
        in_specs=[pl.BlockSpec((NC, ROWB, 16), lambda i: (0, i, 0)),
                  pl.BlockSpec((NC, ROWB, D), lambda i: (0, i, 0)),
                  pl.BlockSpec((ROWB, D), _row_block),
                  pl.BlockSpec((1, D), lambda i: (0, 0)),
                  pl.BlockSpec((D, D), lambda i: (0, 0)),
                  pl.BlockSpec((1, D), lambda i: (0, 0))],
        out_specs=pl.BlockSpec((ROWB, D), _row_block),
        out_shape=jax.ShapeDtypeStruct((N, D), _f32),
    )(degp, p, g, b.reshape(1, D), w, b3.reshape(1, D))


def kernel(x, edge_index, edge_weight, W1, b1, W2, b2, W3, b3):
    x = x.reshape(-1, D).astype(_f32)
    src3 = edge_index[0].astype(_i32).reshape(NW, EPT)
    dst3 = edge_index[1].astype(_i32).reshape(NW, NCHUNK, CHUNK)
    w3 = edge_weight.astype(_f32).reshape(NW, EPT)

    degp = _deg_partials(w3, dst3)
    h1 = _mm(x, W1)
    g1 = _scale(degp, h1)
    p = _agg_partials(g1, src3, dst3, w3)
    g2 = _mid_layer(degp, p, g1, b1, W2)
    q = _agg_partials(g2, src3, dst3, w3)
    return _final_layer(degp, q, g2, b2, W3, b3)


# speedup vs baseline: 2.8899x; 2.8899x over previous
import dataclasses
import functools

import jax
import jax.numpy as jnp
from jax import lax
from jax.experimental import pallas as pl
from jax.experimental.pallas import tpu as pltpu
from jax.experimental.pallas import tpu_sc as plsc

N = 10000
D = 128
E = 320000
NC = 2
NS = 16
NW = NC * NS
EPT = E // NW
CHUNK = 80
NCHUNK = EPT // CHUNK
EPS = E // NS
NCH2 = EPS // CHUNK
GROUPS = CHUNK // 16
DH = D // NC
DQ = 32
DQC = DQ // 16
NPASS = DH // DQ
SLEN = 624
NZB = 104
ROWB = 1000
PBITS = 14
PMASK = (1 << PBITS) - 1

_f32 = jnp.float32
_i32 = jnp.int32


def _vmesh():
    return plsc.VectorSubcoreMesh(core_axis_name="c", subcore_axis_name="s")


def _sc_params():
    cp = pltpu.CompilerParams()
    if "needs_layout_passes" in pltpu.CompilerParams.__dataclass_fields__:
        cp = dataclasses.replace(cp, needs_layout_passes=False)
    return cp


_GATHER_DN = lax.GatherDimensionNumbers(
    offset_dims=(), collapsed_slice_dims=(0,), start_index_map=(0,))


def _bcast_lane(vec, j):
    idx = jnp.full((16, 1), j, _i32)
    return lax.gather(vec, idx, _GATHER_DN, (1,),
                      mode=lax.GatherScatterMode.PROMISE_IN_BOUNDS)


def _store_row(ref, r, sl16, val):
    plsc.store_scatter(ref, [jnp.full((16,), r, _i32),
                             lax.iota(_i32, 16) + 16 * sl16], val)


def _load_row(ref, r, sl16):
    return plsc.load_gather(ref, [jnp.full((16,), r, _i32),
                                  lax.iota(_i32, 16) + 16 * sl16])


def _zero_fill(zb_v, ncols16):
    zero16 = jnp.zeros((16,), _f32)

    @pl.loop(0, NZB)
    def _(r):
        for cc in range(ncols16):
            _store_row(zb_v, r, cc, zero16)


def _zero_acc(zb_v, acc, base):
    for t in range(SLEN // NZB):
        pltpu.sync_copy(zb_v, acc.at[pl.ds(base + t * NZB, NZB)])


def _acc_to_out(acc, out_hbm, cid, sid, base):
    for t in range(SLEN // NZB):
        b = base + t * NZB
        pltpu.sync_copy(acc.at[pl.ds(b, NZB)], out_hbm.at[cid, pl.ds(b, NZB)])

    @pl.when(sid == NS - 1)
    def _():
        pltpu.sync_copy(acc.at[pl.ds(N - 16, 16)],
                        out_hbm.at[cid, pl.ds(N - 16, 16)])


def _unpack_dst(pk_v, dst_v):
    @pl.loop(0, EPS // 16)
    def _(gi):
        v = pk_v[pl.ds(gi * 16, 16)]
        e0 = gi * 16
        plsc.store_scatter(dst_v, [jnp.full((16,), 0, _i32) + e0 // CHUNK,
                                   lax.iota(_i32, 16) + e0 % CHUNK],
                           v & PMASK)


DROWS = 640
DSTRIPE = DROWS // NS


@functools.cache
def _deg_kernel():
    @functools.partial(
        pl.kernel,
        out_type=jax.ShapeDtypeStruct((DROWS, 16), _f32),
        mesh=_vmesh(),
        compiler_params=_sc_params(),
        scratch_types=[
            pltpu.VMEM((2 * CHUNK,), _i32),
            pltpu.VMEM((1, CHUNK), _i32),
            pltpu.VMEM((CHUNK, 16), _f32),
            pltpu.VMEM((DSTRIPE, 16), _f32),
            pltpu.VMEM_SHARED((DROWS, 16), _f32),
        ],
    )
    def k(ed_hbm, out_hbm, ec_v, dstrow_v, rows_v, zb_v, acc):
        cid = lax.axis_index("c")
        sid = lax.axis_index("s")
        zero16 = jnp.zeros((16,), _f32)

        @pl.loop(0, DSTRIPE)
        def _(r):
            _store_row(zb_v, r, 0, zero16)

        pltpu.sync_copy(zb_v, acc.at[pl.ds(sid * DSTRIPE, DSTRIPE)])
        plsc.subcore_barrier()
        lane = lax.iota(_i32, 16)
        ebase = sid * EPS

        iot2 = lax.iota(_i32, 16) * 2

        @pl.loop(0, NCH2)
        def _(c):
            pltpu.sync_copy(
                ed_hbm.at[pl.ds(2 * (ebase + c * CHUNK), 2 * CHUNK)], ec_v)
            for g in range(GROUPS):
                v = plsc.load_gather(ec_v, [iot2 + g * 32])
                wv = plsc.bitcast(plsc.load_gather(ec_v, [iot2 + g * 32 + 1]),
                                  _f32)
                d = v & PMASK
                plsc.store_scatter(dstrow_v,
                                   [jnp.zeros((16,), _i32),
                                    lax.iota(_i32, 16) + g * 16],
                                   lax.shift_right_logical(d, 4))
                dm = d & 15
                for j in range(16):
                    hot = lane == _bcast_lane(dm, j)
                    _store_row(rows_v, g * 16 + j, 0,
                               jnp.where(hot, _bcast_lane(wv, j), 0.0))
            pltpu.sync_copy(rows_v, acc.at[dstrow_v.at[0]], add=True)

        plsc.subcore_barrier()

        @pl.when(cid == 0)
        def _():
            pltpu.sync_copy(acc.at[pl.ds(sid * DSTRIPE, DSTRIPE)],
                            out_hbm.at[pl.ds(sid * DSTRIPE, DSTRIPE)])

    return k


def _deg_partials(ed):
    return _deg_kernel()(ed)


@functools.cache
def _agg_kernel():
    @functools.partial(
        pl.kernel,
        out_type=jax.ShapeDtypeStruct((NC, NPASS, N, DQ), _f32),
        mesh=_vmesh(),
        compiler_params=_sc_params(),
        scratch_types=[
            pltpu.VMEM((2 * CHUNK,), _i32),
            pltpu.VMEM((CHUNK,), _i32),
            pltpu.VMEM((1, CHUNK), _i32),
            pltpu.VMEM((CHUNK,), _f32),
            pltpu.VMEM((CHUNK, D), _f32),
            pltpu.VMEM((CHUNK, DQ), _f32),
            pltpu.VMEM((NZB, DQ), _f32),
            pltpu.VMEM_SHARED((N, DQ), _f32),
            pltpu.SemaphoreType.DMA,
        ],
    )
    def k(g_hbm, ed_hbm, out_hbm,
          ec_v, srcc_v, dstc_v, wc_v, rows_v, q_v, zb_v, acc, sem):
        cid = lax.axis_index("c")
        sid = lax.axis_index("s")
        _zero_fill(zb_v, DQC)
        base = sid * SLEN
        ebase = sid * EPS

        for p in range(NPASS):
            _zero_acc(zb_v, acc, base)

            @pl.when(sid == NS - 1)
            def _():
                pltpu.sync_copy(zb_v.at[pl.ds(0, 16)],
                                acc.at[pl.ds(N - 16, 16)])

            plsc.subcore_barrier()
            coff = cid * (DH // 16) + p * DQC

            iot2 = lax.iota(_i32, 16) * 2

            @pl.loop(0, NCH2)
            def _(c):
                pltpu.sync_copy(
                    ed_hbm.at[pl.ds(2 * (ebase + c * CHUNK), 2 * CHUNK)],
                    ec_v)
                for g_ in range(GROUPS):
                    v = plsc.load_gather(ec_v, [iot2 + g_ * 32])
                    wc_v[pl.ds(g_ * 16, 16)] = plsc.bitcast(
                        plsc.load_gather(ec_v, [iot2 + g_ * 32 + 1]), _f32)
                    srcc_v[pl.ds(g_ * 16, 16)] = lax.shift_right_logical(
                        v, PBITS)
                    plsc.store_scatter(dstc_v,
                                       [jnp.zeros((16,), _i32),
                                        lax.iota(_i32, 16) + g_ * 16],
                                       v & PMASK)
                pltpu.async_copy(g_hbm.at[srcc_v], rows_v, sem).wait()
                for g_ in range(GROUPS):
                    wv = wc_v[pl.ds(g_ * 16, 16)]
                    for j in range(16):
                        sv = _bcast_lane(wv, j)
                        row = g_ * 16 + j
                        for cc in range(DQC):
                            _store_row(q_v, row, cc,
                                       _load_row(rows_v, row, coff + cc) * sv)
                pltpu.sync_copy(q_v, acc.at[dstc_v.at[0]], add=True)

            plsc.subcore_barrier()
            for t in range(SLEN // NZB):
                b = base + t * NZB
                pltpu.sync_copy(acc.at[pl.ds(b, NZB)],
                                out_hbm.at[cid, p, pl.ds(b, NZB)])

            @pl.when(sid == NS - 1)
            def _():
                pltpu.sync_copy(acc.at[pl.ds(N - 16, 16)],
                                out_hbm.at[cid, p, pl.ds(N - 16, 16)])

            plsc.subcore_barrier()

    return k


def _agg_partials(g, ed):
    return _agg_kernel()(g, ed)


def _row_block(i):
    return (i, 0)


def _mm_kernel(x_ref, w_ref, o_ref):
    o_ref[...] = jnp.dot(x_ref[...], w_ref[...], preferred_element_type=_f32)


def _mm(x, w):
    return pl.pallas_call(
        _mm_kernel,
        grid=(N // ROWB,),
        in_specs=[pl.BlockSpec((ROWB, D), _row_block),
                  pl.BlockSpec((D, D), lambda i: (0, 0))],
        out_specs=pl.BlockSpec((ROWB, D), _row_block),
        out_shape=jax.ShapeDtypeStruct((N, D), _f32),
    )(x, w)


def _dinv_of(degp_ref):
    deg = jnp.transpose(degp_ref[0]) + 1.0
    return lax.rsqrt(deg)


def _scale_kernel(degp_ref, h_ref, o_ref):
    o_ref[...] = _dinv_of(degp_ref) * h_ref[...]


def _scale(degp, h):
    return pl.pallas_call(
        _scale_kernel,
        grid=(N // ROWB,),
        in_specs=[pl.BlockSpec((1, 1, ROWB), lambda i: (i, 0, 0)),
                  pl.BlockSpec((ROWB, D), _row_block)],
        out_specs=pl.BlockSpec((ROWB, D), _row_block),
        out_shape=jax.ShapeDtypeStruct((N, D), _f32),
    )(degp, h)


def _mid_kernel(degp_ref, p_ref, g_ref, b_ref, w_ref, o_ref):
    dinv = _dinv_of(degp_ref)
    pfull = jnp.concatenate([p_ref[0, 0], p_ref[0, 1],
                             p_ref[1, 0], p_ref[1, 1]], axis=1)
    z = dinv * (pfull + g_ref[...]) + b_ref[...]
    z = jnp.maximum(z, 0.0)
    o_ref[...] = dinv * jnp.dot(z, w_ref[...], preferred_element_type=_f32)


def _mid_layer(degp, p, g, b, w):
    return pl.pallas_call(
        _mid_kernel,
        grid=(N // ROWB,),
        in_specs=[pl.BlockSpec((1, 1, ROWB), lambda i: (i, 0, 0)),
                  pl.BlockSpec((NC, NPASS, ROWB, DQ),
                               lambda i: (0, 0, i, 0)),
                  pl.BlockSpec((ROWB, D), _row_block),
                  pl.BlockSpec((1, D), lambda i: (0, 0)),
                  pl.BlockSpec((D, D), lambda i: (0, 0))],
        out_specs=pl.BlockSpec((ROWB, D), _row_block),
        out_shape=jax.ShapeDtypeStruct((N, D), _f32),
    )(degp, p, g, b.reshape(1, D), w)


def _final_kernel(degp_ref, p_ref, g_ref, b_ref, w_ref, b3_ref, o_ref):
    dinv = _dinv_of(degp_ref)
    pfull = jnp.concatenate([p_ref[0], p_ref[1]], axis=1)
    z = dinv * (pfull + g_ref[...]) + b_ref[...]
    z = jnp.maximum(z, 0.0)
    o_ref[...] = (jnp.dot(z, w_ref[...], preferred_element_type=_f32)
                  + b3_ref[...])


def _final_layer(degp, p, g, b, w, b3):
    return pl.pallas_call(
        _final_kernel,
        grid=(N // ROWB,),
        in_specs=[pl.BlockSpec((1, 1, ROWB), lambda i: (i, 0, 0)),
                  pl.BlockSpec((NC, ROWB, DH), lambda i: (0, i, 0)),
                  pl.BlockSpec((ROWB, D), _row_block),
                  pl.BlockSpec((1, D), lambda i: (0, 0)),
                  pl.BlockSpec((D, D), lambda i: (0, 0)),
                  pl.BlockSpec((1, D), lambda i: (0, 0))],
        out_specs=pl.BlockSpec((ROWB, D), _row_block),
        out_shape=jax.ShapeDtypeStruct((N, D), _f32),
    )(degp, p, g, b.reshape(1, D), w, b3.reshape(1, D))


def _unscale_kernel(degp_ref, g_ref, b3_ref, o_ref):
    deg = jnp.transpose(degp_ref[0]) + 1.0
    o_ref[...] = jnp.sqrt(deg) * g_ref[...] + b3_ref[...]


def _unscale(degp, g, b3):
    return pl.pallas_call(
        _unscale_kernel,
        grid=(N // ROWB,),
        in_specs=[pl.BlockSpec((1, 1, ROWB), lambda i: (i, 0, 0)),
                  pl.BlockSpec((ROWB, D), _row_block),
                  pl.BlockSpec((1, D), lambda i: (0, 0))],
        out_specs=pl.BlockSpec((ROWB, D), _row_block),
        out_shape=jax.ShapeDtypeStruct((N, D), _f32),
    )(degp, g, b3.reshape(1, D))


def kernel(x, edge_index, edge_weight, W1, b1, W2, b2, W3, b3):
    x = x.reshape(-1, D).astype(_f32)
    src = edge_index[0].astype(_i32)
    dst = edge_index[1].astype(_i32)
    pk = (src << PBITS) | dst
    wbits = lax.bitcast_convert_type(edge_weight.astype(_f32), _i32)
    ed = jnp.stack([pk, wbits], axis=1).reshape(2 * E)

    degr = _deg_partials(ed)
    degp = degr.reshape(DROWS * 16)[:N].reshape(N // ROWB, 1, ROWB)
    h1 = _mm(x, W1)
    g1 = _scale(degp, h1)

    def body(g, params):
        b_l, wn_l = params
        p = _agg_partials(g, ed)
        return _mid_layer(degp, p, g, b_l, wn_l), None

    gfin, _ = lax.scan(body, g1, (jnp.stack([b1, b2]), jnp.stack([W2, W3])))
    return _unscale(degp, gfin, b3)


# speedup vs baseline: 2.8906x; 1.0002x over previous
import dataclasses
import functools

import jax
import jax.numpy as jnp
from jax import lax
from jax.experimental import pallas as pl
from jax.experimental.pallas import tpu as pltpu
from jax.experimental.pallas import tpu_sc as plsc

N = 10000
D = 128
E = 320000
NC = 2
NS = 16
NW = NC * NS
EPT = E // NW
CHUNK = 80
EPS = E // NS
NCH2 = EPS // CHUNK
GROUPS = CHUNK // 16
DH = D // NC
DQ = 32
DQC = DQ // 16
NPASS = DH // DQ
SLEN = 624
NZB = 104
ROWB = 1000
PBITS = 14
PMASK = (1 << PBITS) - 1

_f32 = jnp.float32
_i32 = jnp.int32


def _vmesh():
    return plsc.VectorSubcoreMesh(core_axis_name="c", subcore_axis_name="s")


def _sc_params():
    cp = pltpu.CompilerParams()
    if "needs_layout_passes" in pltpu.CompilerParams.__dataclass_fields__:
        cp = dataclasses.replace(cp, needs_layout_passes=False)
    return cp


_GATHER_DN = lax.GatherDimensionNumbers(
    offset_dims=(), collapsed_slice_dims=(0,), start_index_map=(0,))


def _bcast_lane(vec, j):
    idx = jnp.full((16, 1), j, _i32)
    return lax.gather(vec, idx, _GATHER_DN, (1,),
                      mode=lax.GatherScatterMode.PROMISE_IN_BOUNDS)


def _store_row(ref, r, sl16, val):
    plsc.store_scatter(ref, [jnp.full((16,), r, _i32),
                             lax.iota(_i32, 16) + 16 * sl16], val)


def _load_row(ref, r, sl16):
    return plsc.load_gather(ref, [jnp.full((16,), r, _i32),
                                  lax.iota(_i32, 16) + 16 * sl16])


def _zero_fill(zb_v, ncols16):
    zero16 = jnp.zeros((16,), _f32)

    @pl.loop(0, NZB)
    def _(r):
        for cc in range(ncols16):
            _store_row(zb_v, r, cc, zero16)


def _zero_acc(zb_v, acc, base):
    for t in range(SLEN // NZB):
        pltpu.sync_copy(zb_v, acc.at[pl.ds(base + t * NZB, NZB)])


def _acc_to_out(acc, out_hbm, cid, sid, base):
    for t in range(SLEN // NZB):
        b = base + t * NZB
        pltpu.sync_copy(acc.at[pl.ds(b, NZB)], out_hbm.at[cid, pl.ds(b, NZB)])

    @pl.when(sid == NS - 1)
    def _():
        pltpu.sync_copy(acc.at[pl.ds(N - 16, 16)],
                        out_hbm.at[cid, pl.ds(N - 16, 16)])


DROWS = 640
DSTRIPE = DROWS // NS


@functools.cache
def _deg_kernel():
    @functools.partial(
        pl.kernel,
        out_type=jax.ShapeDtypeStruct((DROWS, 16), _f32),
        mesh=_vmesh(),
        compiler_params=_sc_params(),
        scratch_types=[
            pltpu.VMEM((2 * CHUNK,), _i32),
            pltpu.VMEM((1, CHUNK), _i32),
            pltpu.VMEM((CHUNK, 16), _f32),
            pltpu.VMEM((DSTRIPE, 16), _f32),
            pltpu.VMEM_SHARED((DROWS, 16), _f32),
        ],
    )
    def k(ed_hbm, out_hbm, ec_v, dstrow_v, rows_v, zb_v, acc):
        cid = lax.axis_index("c")
        sid = lax.axis_index("s")
        zero16 = jnp.zeros((16,), _f32)

        @pl.loop(0, DSTRIPE)
        def _(r):
            _store_row(zb_v, r, 0, zero16)

        pltpu.sync_copy(zb_v, acc.at[pl.ds(sid * DSTRIPE, DSTRIPE)])
        plsc.subcore_barrier()
        lane = lax.iota(_i32, 16)
        ebase = sid * EPS

        iot2 = lax.iota(_i32, 16) * 2

        @pl.loop(0, NCH2)
        def _(c):
            pltpu.sync_copy(
                ed_hbm.at[pl.ds(2 * (ebase + c * CHUNK), 2 * CHUNK)], ec_v)
            for g in range(GROUPS):
                v = plsc.load_gather(ec_v, [iot2 + g * 32])
                wv = plsc.bitcast(plsc.load_gather(ec_v, [iot2 + g * 32 + 1]),
                                  _f32)
                d = v & PMASK
                plsc.store_scatter(dstrow_v,
                                   [jnp.zeros((16,), _i32),
                                    lax.iota(_i32, 16) + g * 16],
                                   lax.shift_right_logical(d, 4))
                dm = d & 15
                for j in range(16):
                    hot = lane == _bcast_lane(dm, j)
                    _store_row(rows_v, g * 16 + j, 0,
                               jnp.where(hot, _bcast_lane(wv, j), 0.0))
            pltpu.sync_copy(rows_v, acc.at[dstrow_v.at[0]], add=True)

        plsc.subcore_barrier()

        @pl.when(cid == 0)
        def _():
            pltpu.sync_copy(acc.at[pl.ds(sid * DSTRIPE, DSTRIPE)],
                            out_hbm.at[pl.ds(sid * DSTRIPE, DSTRIPE)])

    return k


def _deg_partials(ed):
    return _deg_kernel()(ed)


@functools.cache
def _agg_kernel():
    @functools.partial(
        pl.kernel,
        out_type=jax.ShapeDtypeStruct((NC, NPASS, N, DQ), _f32),
        mesh=_vmesh(),
        compiler_params=_sc_params(),
        scratch_types=[
            pltpu.VMEM((2 * CHUNK,), _i32),
            pltpu.VMEM((CHUNK,), _i32),
            pltpu.VMEM((1, CHUNK), _i32),
            pltpu.VMEM((CHUNK,), _f32),
            pltpu.VMEM((CHUNK, D), _f32),
            pltpu.VMEM((CHUNK, DQ), _f32),
            pltpu.VMEM((NZB, DQ), _f32),
            pltpu.VMEM_SHARED((N, DQ), _f32),
            pltpu.SemaphoreType.DMA,
        ],
    )
    def k(g_hbm, ed_hbm, out_hbm,
          ec_v, srcc_v, dstc_v, wc_v, rows_v, q_v, zb_v, acc, sem):
        cid = lax.axis_index("c")
        sid = lax.axis_index("s")
        _zero_fill(zb_v, DQC)
        base = sid * SLEN
        ebase = sid * EPS

        for p in range(NPASS):
            _zero_acc(zb_v, acc, base)

            @pl.when(sid == NS - 1)
            def _():
                pltpu.sync_copy(zb_v.at[pl.ds(0, 16)],
                                acc.at[pl.ds(N - 16, 16)])

            plsc.subcore_barrier()
            coff = cid * (DH // 16) + p * DQC

            iot2 = lax.iota(_i32, 16) * 2

            @pl.loop(0, NCH2)
            def _(c):
                pltpu.sync_copy(
                    ed_hbm.at[pl.ds(2 * (ebase + c * CHUNK), 2 * CHUNK)],
                    ec_v)
                for g_ in range(GROUPS):
                    v = plsc.load_gather(ec_v, [iot2 + g_ * 32])
                    wc_v[pl.ds(g_ * 16, 16)] = plsc.bitcast(
                        plsc.load_gather(ec_v, [iot2 + g_ * 32 + 1]), _f32)
                    srcc_v[pl.ds(g_ * 16, 16)] = lax.shift_right_logical(
                        v, PBITS)
                    plsc.store_scatter(dstc_v,
                                       [jnp.zeros((16,), _i32),
                                        lax.iota(_i32, 16) + g_ * 16],
                                       v & PMASK)
                pltpu.async_copy(g_hbm.at[srcc_v], rows_v, sem).wait()
                for g_ in range(GROUPS):
                    wv = wc_v[pl.ds(g_ * 16, 16)]
                    for j in range(16):
                        sv = _bcast_lane(wv, j)
                        row = g_ * 16 + j
                        for cc in range(DQC):
                            _store_row(q_v, row, cc,
                                       _load_row(rows_v, row, coff + cc) * sv)
                pltpu.sync_copy(q_v, acc.at[dstc_v.at[0]], add=True)

            plsc.subcore_barrier()
            for t in range(SLEN // NZB):
                b = base + t * NZB
                pltpu.sync_copy(acc.at[pl.ds(b, NZB)],
                                out_hbm.at[cid, p, pl.ds(b, NZB)])

            @pl.when(sid == NS - 1)
            def _():
                pltpu.sync_copy(acc.at[pl.ds(N - 16, 16)],
                                out_hbm.at[cid, p, pl.ds(N - 16, 16)])

            plsc.subcore_barrier()

    return k


def _agg_partials(g, ed):
    return _agg_kernel()(g, ed)


def _row_block(i):
    return (i, 0)


def _mm_kernel(x_ref, w_ref, o_ref):
    o_ref[...] = jnp.dot(x_ref[...], w_ref[...], preferred_element_type=_f32)


def _mm(x, w):
    return pl.pallas_call(
        _mm_kernel,
        grid=(N // ROWB,),
        in_specs=[pl.BlockSpec((ROWB, D), _row_block),
                  pl.BlockSpec((D, D), lambda i: (0, 0))],
        out_specs=pl.BlockSpec((ROWB, D), _row_block),
        out_shape=jax.ShapeDtypeStruct((N, D), _f32),
    )(x, w)


def _dinv_of(degp_ref):
    deg = jnp.transpose(degp_ref[0]) + 1.0
    return lax.rsqrt(deg)


def _scale_kernel(degp_ref, h_ref, o_ref):
    o_ref[...] = _dinv_of(degp_ref) * h_ref[...]


def _scale(degp, h):
    return pl.pallas_call(
        _scale_kernel,
        grid=(N // ROWB,),
        in_specs=[pl.BlockSpec((1, 1, ROWB), lambda i: (i, 0, 0)),
                  pl.BlockSpec((ROWB, D), _row_block)],
        out_specs=pl.BlockSpec((ROWB, D), _row_block),
        out_shape=jax.ShapeDtypeStruct((N, D), _f32),
    )(degp, h)


def _mid_kernel(degp_ref, p_ref, g_ref, b_ref, w_ref, o_ref):
    dinv = _dinv_of(degp_ref)
    pfull = jnp.concatenate([p_ref[0, 0], p_ref[0, 1],
                             p_ref[1, 0], p_ref[1, 1]], axis=1)
    z = dinv * (pfull + g_ref[...]) + b_ref[...]
    z = jnp.maximum(z, 0.0)
    o_ref[...] = dinv * jnp.dot(z, w_ref[...], preferred_element_type=_f32)


def _mid_layer(degp, p, g, b, w):
    return pl.pallas_call(
        _mid_kernel,
        grid=(N // ROWB,),
        in_specs=[pl.BlockSpec((1, 1, ROWB), lambda i: (i, 0, 0)),
                  pl.BlockSpec((NC, NPASS, ROWB, DQ),
                               lambda i: (0, 0, i, 0)),
                  pl.BlockSpec((ROWB, D), _row_block),
                  pl.BlockSpec((1, D), lambda i: (0, 0)),
                  pl.BlockSpec((D, D), lambda i: (0, 0))],
        out_specs=pl.BlockSpec((ROWB, D), _row_block),
        out_shape=jax.ShapeDtypeStruct((N, D), _f32),
    )(degp, p, g, b.reshape(1, D), w)


def _final_kernel(degp_ref, p_ref, g_ref, b_ref, w_ref, b3_ref, o_ref):
    dinv = _dinv_of(degp_ref)
    pfull = jnp.concatenate([p_ref[0], p_ref[1]], axis=1)
    z = dinv * (pfull + g_ref[...]) + b_ref[...]
    z = jnp.maximum(z, 0.0)
    o_ref[...] = (jnp.dot(z, w_ref[...], preferred_element_type=_f32)
                  + b3_ref[...])


def _final_layer(degp, p, g, b, w, b3):
    return pl.pallas_call(
        _final_kernel,
        grid=(N // ROWB,),
        in_specs=[pl.BlockSpec((1, 1, ROWB), lambda i: (i, 0, 0)),
                  pl.BlockSpec((NC, ROWB, DH), lambda i: (0, i, 0)),
                  pl.BlockSpec((ROWB, D), _row_block),
                  pl.BlockSpec((1, D), lambda i: (0, 0)),
                  pl.BlockSpec((D, D), lambda i: (0, 0)),
                  pl.BlockSpec((1, D), lambda i: (0, 0))],
        out_specs=pl.BlockSpec((ROWB, D), _row_block),
        out_shape=jax.ShapeDtypeStruct((N, D), _f32),
    )(degp, p, g, b.reshape(1, D), w, b3.reshape(1, D))


def _unscale_kernel(degp_ref, g_ref, b3_ref, o_ref):
    deg = jnp.transpose(degp_ref[0]) + 1.0
    o_ref[...] = jnp.sqrt(deg) * g_ref[...] + b3_ref[...]


def _unscale(degp, g, b3):
    return pl.pallas_call(
        _unscale_kernel,
        grid=(N // ROWB,),
        in_specs=[pl.BlockSpec((1, 1, ROWB), lambda i: (i, 0, 0)),
                  pl.BlockSpec((ROWB, D), _row_block),
                  pl.BlockSpec((1, D), lambda i: (0, 0))],
        out_specs=pl.BlockSpec((ROWB, D), _row_block),
        out_shape=jax.ShapeDtypeStruct((N, D), _f32),
    )(degp, g, b3.reshape(1, D))


def kernel(x, edge_index, edge_weight, W1, b1, W2, b2, W3, b3):
    x = x.reshape(-1, D).astype(_f32)
    src = edge_index[0].astype(_i32)
    dst = edge_index[1].astype(_i32)
    pk = (src << PBITS) | dst
    wbits = lax.bitcast_convert_type(edge_weight.astype(_f32), _i32)
    ed = jnp.stack([pk, wbits], axis=1).reshape(2 * E)

    degr = _deg_partials(ed)
    degp = degr.reshape(DROWS * 16)[:N].reshape(N // ROWB, 1, ROWB)
    h1 = _mm(x, W1)
    g1 = _scale(degp, h1)

    def body(g, params):
        b_l, wn_l = params
        p = _agg_partials(g, ed)
        return _mid_layer(degp, p, g, b_l, wn_l), None

    gfin, _ = lax.scan(body, g1, (jnp.stack([b1, b2]), jnp.stack([W2, W3])))
    return _unscale(degp, gfin, b3)
